# Initial kernel scaffold; baseline (speedup 1.0000x reference)
import jax
import jax.numpy as jnp
from jax.experimental import pallas as pl


def kernel(X):
    raise NotImplementedError("write your pallas kernel here")


# speedup vs baseline: 8.4178x; 8.4178x over previous
import functools

import jax
import jax.numpy as jnp
from jax import lax
from jax.experimental import pallas as pl
from jax.experimental.pallas import tpu as pltpu
from jax.experimental.pallas import tpu_sc as plsc

B = 2
N = 4096
DIM = 32
KNN = 10
ROWS = B * N
CHUNK = 128
NCH = N // CHUNK
RB = 512
LANES = 16
NW = 32
RPT = ROWS // NW
CAPL = 32


def _dist_body(xr_ref, xa_ref, d2_ref, cm_ref):
    xr = xr_ref[0]
    xa = xa_ref[0]
    g = lax.dot_general(
        xr, xa, (((1,), (1,)), ((), ())),
        preferred_element_type=jnp.float32,
        precision=lax.Precision.HIGHEST,
    )
    x2r = jnp.sum(xr * xr, axis=-1)
    x2a = jnp.sum(xa * xa, axis=-1)
    d2 = x2r[:, None] + x2a[None, :] - 2.0 * g
    d2_ref[...] = d2
    cm_ref[...] = jnp.min(d2.reshape(RB, NCH, CHUNK), axis=-1)


def _distances(X):
    return pl.pallas_call(
        _dist_body,
        grid=(B, N // RB),
        in_specs=[
            pl.BlockSpec((1, RB, DIM), lambda b, i: (b, i, 0)),
            pl.BlockSpec((1, N, DIM), lambda b, i: (b, 0, 0)),
        ],
        out_specs=[
            pl.BlockSpec((RB, N), lambda b, i: (b * (N // RB) + i, 0)),
            pl.BlockSpec((RB, NCH), lambda b, i: (b * (N // RB) + i, 0)),
        ],
        out_shape=[
            jax.ShapeDtypeStruct((ROWS, N), jnp.float32),
            jax.ShapeDtypeStruct((ROWS, NCH), jnp.float32),
        ],
    )(X, X)


def _sc_body(d2_hbm, cm_hbm, out_hbm, rowbuf, cmbuf, candbuf, staging):
    nc = 2
    wid = lax.axis_index("s") * nc + lax.axis_index("c")
    base = wid * RPT
    lane = lax.iota(jnp.int32, 16)
    inf = jnp.float32(jnp.inf)

    pltpu.sync_copy(cm_hbm.at[pl.ds(base * NCH, RPT * NCH)], cmbuf)

    def sort_asc(v):
        k, _ = plsc.sort_key_val(v, v)
        return k

    def sort_desc(v):
        k, _ = plsc.sort_key_val(v, v, descending=True)
        return k

    def do_row(jr, _):
        r = base + jr
        pltpu.sync_copy(d2_hbm.at[r], rowbuf)

        a = cmbuf[pl.ds(jr * NCH, 16)]
        b = cmbuf[pl.ds(jr * NCH + 16, 16)]
        low16 = sort_asc(jnp.minimum(sort_asc(a), sort_desc(b)))
        t = jnp.max(jnp.where(lane == KNN, low16, -inf), axis=0)

        def scan_step(j, percount):
            v = rowbuf[pl.ds(j * 16, 16)]
            m = v <= t
            idx = lane * CAPL + jnp.minimum(percount, CAPL - 1)
            plsc.store_scatter(candbuf, [idx], v,
                               mask=m & (percount < CAPL))
            return percount + jnp.where(m, 1, 0)

        percount = lax.fori_loop(0, N // 16, scan_step,
                                 jnp.zeros((16,), jnp.int32), unroll=8)

        nslots = jnp.minimum(jnp.max(percount, axis=0), CAPL)

        def merge_step(j, acc):
            g = plsc.load_gather(candbuf, [lane * CAPL + j])
            g = jnp.where(percount > j, g, inf)
            return sort_asc(jnp.minimum(acc, sort_desc(g)))

        top = lax.fori_loop(0, nslots, merge_step, jnp.full((16,), inf))
        staging[jr, :] = top
        return 0

    lax.fori_loop(0, RPT, do_row, 0)
    pltpu.sync_copy(staging, out_hbm.at[pl.ds(base, RPT)])


def _sc_topk(d2, cm):
    cm_flat = cm.reshape(ROWS * NCH)
    mesh = plsc.VectorSubcoreMesh(core_axis_name="c", subcore_axis_name="s")
    f = pl.kernel(
        _sc_body,
        out_type=jax.ShapeDtypeStruct((ROWS, 16), jnp.float32),
        mesh=mesh,
        compiler_params=pltpu.CompilerParams(needs_layout_passes=False),
        scratch_types=[
            pltpu.VMEM((N,), jnp.float32),
            pltpu.VMEM((RPT * NCH,), jnp.float32),
            pltpu.VMEM((16 * CAPL,), jnp.float32),
            pltpu.VMEM((RPT, 16), jnp.float32),
        ],
    )
    return f(d2, cm_flat)


def _estimator_body(top_ref, out_ref):
    top = top_ref[...]
    lane = lax.broadcasted_iota(jnp.int32, (ROWS, 16), 1)
    w = jnp.where(lane == KNN, jnp.float32(KNN - 1),
                  jnp.where((lane >= 1) & (lane <= KNN - 1),
                            jnp.float32(-1.0), jnp.float32(0.0)))
    x = jnp.maximum(top, jnp.float32(1e-12))
    x = jnp.where(w != 0.0, x, jnp.float32(1.0))
    s = 0.5 * jnp.log(x) * w
    srow = jnp.sum(s, axis=1, keepdims=True)
    rowid = lax.broadcasted_iota(jnp.int32, (ROWS, 1), 0)
    t0 = jnp.sum(jnp.where(rowid < N, srow, 0.0))
    t1 = jnp.sum(jnp.where(rowid >= N, srow, 0.0))
    bi = lax.broadcasted_iota(jnp.int32, (B, 1), 0)
    tot = jnp.where(bi == 0, t0, t1)
    out_ref[...] = jnp.float32(KNN - 1) * N / tot


def _estimator(top16):
    out = pl.pallas_call(
        _estimator_body,
        out_shape=jax.ShapeDtypeStruct((B, 1), jnp.float32),
    )(top16)
    return out[:, 0]


def kernel(X):
    d2, cm = _distances(X)
    top16 = _sc_topk(d2, cm)
    return _estimator(top16)


# speedup vs baseline: 10.7861x; 1.2814x over previous
import functools

import jax
import jax.numpy as jnp
from jax import lax
from jax.experimental import pallas as pl
from jax.experimental.pallas import tpu as pltpu
from jax.experimental.pallas import tpu_sc as plsc

B = 2
N = 4096
DIM = 32
KNN = 10
ROWS = B * N
CHUNK = 128
NCH = N // CHUNK
RB = 512
LANES = 16
NW = 32
RPT = ROWS // NW
CAPL = 32


def _dist_body(xr_ref, xa_ref, d2_ref, cm_ref):
    xr = xr_ref[0]
    xa = xa_ref[0]
    g = lax.dot_general(
        xr, xa, (((1,), (1,)), ((), ())),
        preferred_element_type=jnp.float32,
        precision=lax.Precision.HIGHEST,
    )
    x2r = jnp.sum(xr * xr, axis=-1)
    x2a = jnp.sum(xa * xa, axis=-1)
    d2 = x2r[:, None] + x2a[None, :] - 2.0 * g
    d2_ref[...] = d2
    cm_ref[...] = jnp.min(d2.reshape(RB, NCH, CHUNK), axis=-1)


def _distances(X):
    return pl.pallas_call(
        _dist_body,
        grid=(B, N // RB),
        in_specs=[
            pl.BlockSpec((1, RB, DIM), lambda b, i: (b, i, 0)),
            pl.BlockSpec((1, N, DIM), lambda b, i: (b, 0, 0)),
        ],
        out_specs=[
            pl.BlockSpec((RB, N), lambda b, i: (b * (N // RB) + i, 0)),
            pl.BlockSpec((RB, NCH), lambda b, i: (b * (N // RB) + i, 0)),
        ],
        out_shape=[
            jax.ShapeDtypeStruct((ROWS, N), jnp.float32),
            jax.ShapeDtypeStruct((ROWS, NCH), jnp.float32),
        ],
    )(X, X)


ROWBLK = 8
NBLK = RPT // ROWBLK


def _sc_body(d2_hbm, cm_hbm, out_hbm, rowbuf0, rowbuf1, cmbuf, candbuf,
             staging, sem0, sem1):
    nc = 2
    wid = lax.axis_index("s") * nc + lax.axis_index("c")
    base = wid * RPT
    lane = lax.iota(jnp.int32, 16)
    inf = jnp.float32(jnp.inf)

    pltpu.sync_copy(cm_hbm.at[pl.ds(base * NCH, RPT * NCH)], cmbuf)

    def sort_asc(v):
        k, _ = plsc.sort_key_val(v, v)
        return k

    def sort_desc(v):
        k, _ = plsc.sort_key_val(v, v, descending=True)
        return k

    def do_row(jr, rowbuf, lr):
        a = cmbuf[pl.ds(jr * NCH, 16)]
        b = cmbuf[pl.ds(jr * NCH + 16, 16)]
        low16 = sort_asc(jnp.minimum(sort_asc(a), sort_desc(b)))
        t = jnp.max(jnp.where(lane == KNN, low16, -inf), axis=0)

        def scan_step(j, percount):
            v = rowbuf[lr, pl.ds(j * 16, 16)]
            m = v <= t
            idx = lane * CAPL + jnp.minimum(percount, CAPL - 1)
            plsc.store_scatter(candbuf, [idx], v,
                               mask=m & (percount < CAPL))
            return percount + jnp.where(m, 1, 0)

        percount = lax.fori_loop(0, N // 16, scan_step,
                                 jnp.zeros((16,), jnp.int32), unroll=8)

        nslots = jnp.minimum(jnp.max(percount, axis=0), CAPL)

        def merge_step(j, acc):
            g = plsc.load_gather(candbuf, [lane * CAPL + j])
            g = jnp.where(percount > j, g, inf)
            return sort_asc(jnp.minimum(acc, sort_desc(g)))

        top = lax.fori_loop(0, nslots, merge_step, jnp.full((16,), inf))
        staging[jr, :] = top

    def start_blk(blk, buf, sem):
        pltpu.async_copy(d2_hbm.at[pl.ds(base + blk * ROWBLK, ROWBLK)],
                         buf, sem)

    def wait_blk(buf, sem):
        pltpu.make_async_copy(d2_hbm.at[pl.ds(0, ROWBLK)], buf, sem).wait()

    def do_blk(blk, rowbuf):
        def body(lr, _):
            do_row(blk * ROWBLK + lr, rowbuf, lr)
            return 0
        lax.fori_loop(0, ROWBLK, body, 0)

    start_blk(0, rowbuf0, sem0)

    def pair(g, _):
        start_blk(2 * g + 1, rowbuf1, sem1)
        wait_blk(rowbuf0, sem0)
        do_blk(2 * g, rowbuf0)

        @pl.when(g < NBLK // 2 - 1)
        def _():
            start_blk(2 * g + 2, rowbuf0, sem0)

        wait_blk(rowbuf1, sem1)
        do_blk(2 * g + 1, rowbuf1)
        return 0

    lax.fori_loop(0, NBLK // 2, pair, 0)
    pltpu.sync_copy(staging, out_hbm.at[pl.ds(base, RPT)])


def _sc_topk(d2, cm):
    cm_flat = cm.reshape(ROWS * NCH)
    mesh = plsc.VectorSubcoreMesh(core_axis_name="c", subcore_axis_name="s")
    f = pl.kernel(
        _sc_body,
        out_type=jax.ShapeDtypeStruct((ROWS, 16), jnp.float32),
        mesh=mesh,
        compiler_params=pltpu.CompilerParams(needs_layout_passes=False),
        scratch_types=[
            pltpu.VMEM((ROWBLK, N), jnp.float32),
            pltpu.VMEM((ROWBLK, N), jnp.float32),
            pltpu.VMEM((RPT * NCH,), jnp.float32),
            pltpu.VMEM((16 * CAPL,), jnp.float32),
            pltpu.VMEM((RPT, 16), jnp.float32),
            pltpu.SemaphoreType.DMA,
            pltpu.SemaphoreType.DMA,
        ],
    )
    return f(d2, cm_flat)


def _estimator_body(top_ref, out_ref):
    top = top_ref[...]
    lane = lax.broadcasted_iota(jnp.int32, (ROWS, 16), 1)
    w = jnp.where(lane == KNN, jnp.float32(KNN - 1),
                  jnp.where((lane >= 1) & (lane <= KNN - 1),
                            jnp.float32(-1.0), jnp.float32(0.0)))
    x = jnp.maximum(top, jnp.float32(1e-12))
    x = jnp.where(w != 0.0, x, jnp.float32(1.0))
    s = 0.5 * jnp.log(x) * w
    srow = jnp.sum(s, axis=1, keepdims=True)
    rowid = lax.broadcasted_iota(jnp.int32, (ROWS, 1), 0)
    t0 = jnp.sum(jnp.where(rowid < N, srow, 0.0))
    t1 = jnp.sum(jnp.where(rowid >= N, srow, 0.0))
    bi = lax.broadcasted_iota(jnp.int32, (B, 1), 0)
    tot = jnp.where(bi == 0, t0, t1)
    out_ref[...] = jnp.float32(KNN - 1) * N / tot


def _estimator(top16):
    out = pl.pallas_call(
        _estimator_body,
        out_shape=jax.ShapeDtypeStruct((B, 1), jnp.float32),
    )(top16)
    return out[:, 0]


def kernel(X):
    d2, cm = _distances(X)
    top16 = _sc_topk(d2, cm)
    return _estimator(top16)


# speedup vs baseline: 24.3422x; 2.2568x over previous
import functools

import jax
import jax.numpy as jnp
from jax import lax
from jax.experimental import pallas as pl
from jax.experimental.pallas import tpu as pltpu
from jax.experimental.pallas import tpu_sc as plsc

B = 2
N = 4096
DIM = 32
KNN = 10
ROWS = B * N
CHUNK = 128
NCH = N // CHUNK
RB = 512
LANES = 16
NW = 32
RPT = ROWS // NW
CAPL = 32


def _dist_body(xr_ref, xa_ref, d2_ref, cm_ref):
    xr = xr_ref[0]
    xa = xa_ref[0]
    g = lax.dot_general(
        xr, xa, (((1,), (1,)), ((), ())),
        preferred_element_type=jnp.float32,
        precision=lax.Precision.HIGHEST,
    )
    x2r = jnp.sum(xr * xr, axis=-1)
    x2a = jnp.sum(xa * xa, axis=-1)
    d2 = x2r[:, None] + x2a[None, :] - 2.0 * g
    d2_ref[...] = d2
    cm_ref[...] = jnp.min(d2.reshape(RB, NCH, CHUNK), axis=-1)


def _distances(X):
    return pl.pallas_call(
        _dist_body,
        grid=(B, N // RB),
        in_specs=[
            pl.BlockSpec((1, RB, DIM), lambda b, i: (b, i, 0)),
            pl.BlockSpec((1, N, DIM), lambda b, i: (b, 0, 0)),
        ],
        out_specs=[
            pl.BlockSpec((RB, N), lambda b, i: (b * (N // RB) + i, 0)),
            pl.BlockSpec((RB, NCH), lambda b, i: (b * (N // RB) + i, 0)),
        ],
        out_shape=[
            jax.ShapeDtypeStruct((ROWS, N), jnp.float32),
            jax.ShapeDtypeStruct((ROWS, NCH), jnp.float32),
        ],
    )(X, X)


ROWBLK = 8
NBLK = RPT // ROWBLK


def _sc_body(d2_hbm, cm_hbm, out_hbm, rowbuf0, rowbuf1, cmbuf, candbuf,
             staging, sem0, sem1):
    nc = 2
    wid = lax.axis_index("s") * nc + lax.axis_index("c")
    base = wid * RPT
    lane = lax.iota(jnp.int32, 16)
    inf = jnp.float32(jnp.inf)

    pltpu.sync_copy(cm_hbm.at[pl.ds(base * NCH, RPT * NCH)], cmbuf)

    def sort_asc(v):
        k, _ = plsc.sort_key_val(v, v)
        return k

    def sort_desc(v):
        k, _ = plsc.sort_key_val(v, v, descending=True)
        return k

    def do_row(jr, rowbuf, lr):
        a = cmbuf[pl.ds(jr * NCH, 16)]
        b = cmbuf[pl.ds(jr * NCH + 16, 16)]
        low16 = sort_asc(jnp.minimum(sort_asc(a), sort_desc(b)))
        t = jnp.max(jnp.where(lane == KNN, low16, -inf), axis=0)

        @plsc.parallel_loop(0, N // 16, unroll=8,
                            carry=jnp.zeros((16,), jnp.int32))
        def scan_step(j, percount):
            v = rowbuf[lr, pl.ds(j * 16, 16)]
            m = v <= t
            idx = lane * CAPL + jnp.minimum(percount, CAPL - 1)
            plsc.store_scatter(candbuf, [idx], v,
                               mask=m & (percount < CAPL))
            return percount + jnp.where(m, 1, 0)

        percount = scan_step

        nslots = jnp.minimum(jnp.max(percount, axis=0), CAPL)

        def merge_step(j, acc):
            g = plsc.load_gather(candbuf, [lane * CAPL + j])
            g = jnp.where(percount > j, g, inf)
            return sort_asc(jnp.minimum(acc, sort_desc(g)))

        top = lax.fori_loop(0, nslots, merge_step, jnp.full((16,), inf))
        staging[jr, :] = top

    def start_blk(blk, buf, sem):
        pltpu.async_copy(d2_hbm.at[pl.ds(base + blk * ROWBLK, ROWBLK)],
                         buf, sem)

    def wait_blk(buf, sem):
        pltpu.make_async_copy(d2_hbm.at[pl.ds(0, ROWBLK)], buf, sem).wait()

    def do_blk(blk, rowbuf):
        def body(lr, _):
            do_row(blk * ROWBLK + lr, rowbuf, lr)
            return 0
        lax.fori_loop(0, ROWBLK, body, 0)

    start_blk(0, rowbuf0, sem0)

    def pair(g, _):
        start_blk(2 * g + 1, rowbuf1, sem1)
        wait_blk(rowbuf0, sem0)
        do_blk(2 * g, rowbuf0)

        @pl.when(g < NBLK // 2 - 1)
        def _():
            start_blk(2 * g + 2, rowbuf0, sem0)

        wait_blk(rowbuf1, sem1)
        do_blk(2 * g + 1, rowbuf1)
        return 0

    lax.fori_loop(0, NBLK // 2, pair, 0)
    pltpu.sync_copy(staging, out_hbm.at[pl.ds(base, RPT)])


def _sc_topk(d2, cm):
    cm_flat = cm.reshape(ROWS * NCH)
    mesh = plsc.VectorSubcoreMesh(core_axis_name="c", subcore_axis_name="s")
    f = pl.kernel(
        _sc_body,
        out_type=jax.ShapeDtypeStruct((ROWS, 16), jnp.float32),
        mesh=mesh,
        compiler_params=pltpu.CompilerParams(needs_layout_passes=False),
        scratch_types=[
            pltpu.VMEM((ROWBLK, N), jnp.float32),
            pltpu.VMEM((ROWBLK, N), jnp.float32),
            pltpu.VMEM((RPT * NCH,), jnp.float32),
            pltpu.VMEM((16 * CAPL,), jnp.float32),
            pltpu.VMEM((RPT, 16), jnp.float32),
            pltpu.SemaphoreType.DMA,
            pltpu.SemaphoreType.DMA,
        ],
    )
    return f(d2, cm_flat)


def _estimator_body(top_ref, out_ref):
    top = top_ref[...]
    lane = lax.broadcasted_iota(jnp.int32, (ROWS, 16), 1)
    w = jnp.where(lane == KNN, jnp.float32(KNN - 1),
                  jnp.where((lane >= 1) & (lane <= KNN - 1),
                            jnp.float32(-1.0), jnp.float32(0.0)))
    x = jnp.maximum(top, jnp.float32(1e-12))
    x = jnp.where(w != 0.0, x, jnp.float32(1.0))
    s = 0.5 * jnp.log(x) * w
    srow = jnp.sum(s, axis=1, keepdims=True)
    rowid = lax.broadcasted_iota(jnp.int32, (ROWS, 1), 0)
    t0 = jnp.sum(jnp.where(rowid < N, srow, 0.0))
    t1 = jnp.sum(jnp.where(rowid >= N, srow, 0.0))
    bi = lax.broadcasted_iota(jnp.int32, (B, 1), 0)
    tot = jnp.where(bi == 0, t0, t1)
    out_ref[...] = jnp.float32(KNN - 1) * N / tot


def _estimator(top16):
    out = pl.pallas_call(
        _estimator_body,
        out_shape=jax.ShapeDtypeStruct((B, 1), jnp.float32),
    )(top16)
    return out[:, 0]


def kernel(X):
    d2, cm = _distances(X)
    top16 = _sc_topk(d2, cm)
    return _estimator(top16)
